# baseline (speedup 1.0000x reference)
import jax
import jax.numpy as jnp
from jax import lax
from jax.experimental import pallas as pl
from jax.experimental.pallas import tpu as pltpu

_N_PRIORS = 8732
_SUB = 8
_LANES = 1092
_P_PAD = _SUB * _LANES
_N_CLASSES = 81
_BATCH = 32
_N_OBJS = 16
_THRESHOLD = 0.5
_NEG_POS_RATIO = 3


def _fused_kernel(tb_ref, tc_ref, pc_ref, pbt_ref, s_ref,
                  neg_ref, np_ref, ls_ref, cp_ref):
    pcx = pc_ref[0]
    pcy = pc_ref[1]
    pw = pc_ref[2]
    ph = pc_ref[3]
    xmin = pcx - pw * 0.5
    ymin = pcy - ph * 0.5
    xmax = pcx + pw * 0.5
    ymax = pcy + ph * 0.5
    area_p = (xmax - xmin) * (ymax - ymin)

    shape = (_SUB, _LANES)
    idx = (lax.broadcasted_iota(jnp.int32, shape, 0) * _LANES
           + lax.broadcasted_iota(jnp.int32, shape, 1))
    maxov = jnp.full(shape, -1.0, jnp.float32)
    argobj = jnp.zeros(shape, jnp.int32)
    best_idx = []
    coords = []
    for o in range(_N_OBJS):
        x0 = tb_ref[0, o, 0]
        y0 = tb_ref[0, o, 1]
        x1 = tb_ref[0, o, 2]
        y1 = tb_ref[0, o, 3]
        coords.append((x0, y0, x1, y1))
        iw = jnp.maximum(jnp.minimum(x1, xmax) - jnp.maximum(x0, xmin), 0.0)
        ih = jnp.maximum(jnp.minimum(y1, ymax) - jnp.maximum(y0, ymin), 0.0)
        inter = iw * ih
        union = (x1 - x0) * (y1 - y0) + area_p - inter
        ov = inter / union
        upd = ov > maxov
        argobj = jnp.where(upd, o, argobj)
        maxov = jnp.maximum(maxov, ov)
        m = jnp.max(ov)
        best_idx.append(jnp.min(jnp.where(ov == m, idx, jnp.int32(2**30))))
    for o in range(_N_OBJS):
        sel = idx == best_idx[o]
        argobj = jnp.where(sel, o, argobj)
        maxov = jnp.where(sel, 1.0, maxov)

    cls = jnp.zeros(shape, jnp.float32)
    tcx = jnp.zeros(shape, jnp.float32)
    tcy = jnp.zeros(shape, jnp.float32)
    tw = jnp.ones(shape, jnp.float32)
    th = jnp.ones(shape, jnp.float32)
    for o in range(_N_OBJS):
        sel = argobj == o
        x0, y0, x1, y1 = coords[o]
        cls = jnp.where(sel, tc_ref[0, 0, o].astype(jnp.float32), cls)
        tcx = jnp.where(sel, (x0 + x1) * 0.5, tcx)
        tcy = jnp.where(sel, (y0 + y1) * 0.5, tcy)
        tw = jnp.where(sel, x1 - x0, tw)
        th = jnp.where(sel, y1 - y0, th)
    cls = jnp.where(maxov < _THRESHOLD, 0.0, cls)
    posf = (cls != 0.0).astype(jnp.float32)

    g_cx = (tcx - pcx) / (pw / 10.0)
    g_cy = (tcy - pcy) / (ph / 10.0)
    g_w = jnp.log(tw / pw) * 5.0
    g_h = jnp.log(th / ph) * 5.0
    l1 = (jnp.abs(pbt_ref[0, 0] - g_cx)
          + jnp.abs(pbt_ref[0, 1] - g_cy)
          + jnp.abs(pbt_ref[0, 2] - g_w)
          + jnp.abs(pbt_ref[0, 3] - g_h))
    ls_ref[0, 0, 0] = jnp.sum(l1 * posf)
    np_ref[0, 0, 0] = jnp.sum(posf)

    cls_pos = jnp.zeros((), jnp.float32)
    for r in range(_SUB):
        r0 = r * _LANES
        rows = min(_LANES, _N_PRIORS - r0)
        st = s_ref[0, r0:r0 + rows, :].T
        m = jnp.max(st, axis=0, keepdims=True)
        e = jnp.exp(st - m)
        lse = m + jnp.log(jnp.sum(e, axis=0, keepdims=True))
        sio = lax.broadcasted_iota(jnp.int32, (_N_CLASSES, rows), 0)
        ci = cls[r:r + 1, 0:rows]
        strue = jnp.sum(jnp.where(sio == ci.astype(jnp.int32), st, 0.0),
                        axis=0, keepdims=True)
        ce = lse - strue
        pf = posf[r:r + 1, 0:rows]
        cls_pos = cls_pos + jnp.sum(ce * pf)
        neg_ref[0, 0, r0:r0 + rows] = (ce * (1.0 - pf))[0]
    cp_ref[0, 0, 0] = cls_pos


def _mine_kernel(neg_ref, np_ref, ls_ref, cp_ref, out_ref):
    neg = neg_ref[...]
    npos = jnp.sum(np_ref[...])
    cls_pos = jnp.sum(cp_ref[...])
    k = jnp.minimum(jnp.float32(_NEG_POS_RATIO) * npos, jnp.float32(_N_PRIORS))

    lo = jnp.zeros((_BATCH, 1), jnp.int32)
    hi = jnp.full((_BATCH, 1), jnp.int32(0x7F800000))

    def body(_, carry):
        lo, hi = carry
        mid = lo + ((hi - lo) >> 1)
        midf = lax.bitcast_convert_type(mid, jnp.float32)
        cnt = jnp.sum((neg >= midf).astype(jnp.float32), axis=1, keepdims=True)
        p = cnt >= k
        return jnp.where(p, mid, lo), jnp.where(p, hi, mid)

    lo, hi = lax.fori_loop(0, 31, body, (lo, hi))
    v = lax.bitcast_convert_type(lo, jnp.float32)
    gt = (neg > v).astype(jnp.float32)
    cnt_gt = jnp.sum(gt, axis=1, keepdims=True)
    sum_gt = jnp.sum(neg * gt, axis=1, keepdims=True)
    cls_hard = jnp.sum(sum_gt + (k - cnt_gt) * v)

    loc_loss = jnp.sum(ls_ref[...]) / (npos * 4.0)
    out_ref[0, 0] = loc_loss + (cls_pos + cls_hard) / npos


def kernel(pred_boxes, pred_scores, true_boxes, true_classes, pboxes):
    b, p = _BATCH, _N_PRIORS
    pad = _P_PAD - p
    pad_vals = jnp.tile(jnp.array([[50.0], [50.0], [1.0], [1.0]],
                                  jnp.float32), (1, pad))
    pc_t = jnp.concatenate([jnp.transpose(pboxes, (1, 0)), pad_vals],
                           axis=1).reshape(4, _SUB, _LANES)
    pb_t = jnp.concatenate(
        [jnp.transpose(pred_boxes, (0, 2, 1)),
         jnp.zeros((b, 4, pad), jnp.float32)], axis=2
    ).reshape(b, 4, _SUB, _LANES)

    neg, npos, locsum, clspos = pl.pallas_call(
        _fused_kernel,
        grid=(b,),
        in_specs=[
            pl.BlockSpec((1, _N_OBJS, 4), lambda i: (i, 0, 0),
                         memory_space=pltpu.SMEM),
            pl.BlockSpec((1, 1, _N_OBJS), lambda i: (i, 0, 0),
                         memory_space=pltpu.SMEM),
            pl.BlockSpec((4, _SUB, _LANES), lambda i: (0, 0, 0)),
            pl.BlockSpec((1, 4, _SUB, _LANES), lambda i: (i, 0, 0, 0)),
            pl.BlockSpec((1, p, _N_CLASSES), lambda i: (i, 0, 0)),
        ],
        out_specs=[
            pl.BlockSpec((1, 1, p), lambda i: (i, 0, 0)),
            pl.BlockSpec((1, 1, 1), lambda i: (i, 0, 0),
                         memory_space=pltpu.SMEM),
            pl.BlockSpec((1, 1, 1), lambda i: (i, 0, 0),
                         memory_space=pltpu.SMEM),
            pl.BlockSpec((1, 1, 1), lambda i: (i, 0, 0),
                         memory_space=pltpu.SMEM),
        ],
        compiler_params=pltpu.CompilerParams(
            dimension_semantics=("arbitrary",)),
        out_shape=[
            jax.ShapeDtypeStruct((b, 1, p), jnp.float32),
            jax.ShapeDtypeStruct((b, 1, 1), jnp.float32),
            jax.ShapeDtypeStruct((b, 1, 1), jnp.float32),
            jax.ShapeDtypeStruct((b, 1, 1), jnp.float32),
        ],
    )(true_boxes, true_classes[:, None], pc_t, pb_t, pred_scores)

    loss = pl.pallas_call(
        _mine_kernel,
        in_specs=[
            pl.BlockSpec((b, p), lambda: (0, 0)),
            pl.BlockSpec((b, 1, 1), lambda: (0, 0, 0)),
            pl.BlockSpec((b, 1, 1), lambda: (0, 0, 0)),
            pl.BlockSpec((b, 1, 1), lambda: (0, 0, 0)),
        ],
        out_specs=pl.BlockSpec((1, 1), lambda: (0, 0),
                               memory_space=pltpu.SMEM),
        out_shape=jax.ShapeDtypeStruct((1, 1), jnp.float32),
    )(neg.reshape(b, p), npos, locsum, clspos)

    return loss[0, 0]


# speedup vs baseline: 1.0026x; 1.0026x over previous
import jax
import jax.numpy as jnp
from jax import lax
from jax.experimental import pallas as pl
from jax.experimental.pallas import tpu as pltpu

_N_PRIORS = 8732
_SUB = 8
_LANES = 1092
_P_PAD = _SUB * _LANES
_N_CLASSES = 81
_BATCH = 32
_N_OBJS = 16
_THRESHOLD = 0.5
_NEG_POS_RATIO = 3


def _fused_kernel(tb_ref, tc_ref, pc_ref, pbt_ref, s_ref,
                  neg_ref, np_ref, ls_ref, cp_ref):
    pcx = pc_ref[0]
    pcy = pc_ref[1]
    pw = pc_ref[2]
    ph = pc_ref[3]
    xmin = pcx - pw * 0.5
    ymin = pcy - ph * 0.5
    xmax = pcx + pw * 0.5
    ymax = pcy + ph * 0.5
    area_p = (xmax - xmin) * (ymax - ymin)

    shape = (_SUB, _LANES)
    idx = (lax.broadcasted_iota(jnp.int32, shape, 0) * _LANES
           + lax.broadcasted_iota(jnp.int32, shape, 1))
    maxov = jnp.full(shape, -1.0, jnp.float32)
    argobj = jnp.zeros(shape, jnp.int32)
    best_idx = []
    coords = []
    for o in range(_N_OBJS):
        x0 = tb_ref[0, o, 0]
        y0 = tb_ref[0, o, 1]
        x1 = tb_ref[0, o, 2]
        y1 = tb_ref[0, o, 3]
        coords.append((x0, y0, x1, y1))
        iw = jnp.maximum(jnp.minimum(x1, xmax) - jnp.maximum(x0, xmin), 0.0)
        ih = jnp.maximum(jnp.minimum(y1, ymax) - jnp.maximum(y0, ymin), 0.0)
        inter = iw * ih
        union = (x1 - x0) * (y1 - y0) + area_p - inter
        ov = inter / union
        upd = ov > maxov
        argobj = jnp.where(upd, o, argobj)
        maxov = jnp.maximum(maxov, ov)
        m = jnp.max(ov)
        best_idx.append(jnp.min(jnp.where(ov == m, idx, jnp.int32(2**30))))
    for o in range(_N_OBJS):
        sel = idx == best_idx[o]
        argobj = jnp.where(sel, o, argobj)
        maxov = jnp.where(sel, 1.0, maxov)

    cls = jnp.zeros(shape, jnp.float32)
    tcx = jnp.zeros(shape, jnp.float32)
    tcy = jnp.zeros(shape, jnp.float32)
    tw = jnp.ones(shape, jnp.float32)
    th = jnp.ones(shape, jnp.float32)
    for o in range(_N_OBJS):
        sel = argobj == o
        x0, y0, x1, y1 = coords[o]
        cls = jnp.where(sel, tc_ref[0, 0, o].astype(jnp.float32), cls)
        tcx = jnp.where(sel, (x0 + x1) * 0.5, tcx)
        tcy = jnp.where(sel, (y0 + y1) * 0.5, tcy)
        tw = jnp.where(sel, x1 - x0, tw)
        th = jnp.where(sel, y1 - y0, th)
    cls = jnp.where(maxov < _THRESHOLD, 0.0, cls)
    posf = (cls != 0.0).astype(jnp.float32)

    g_cx = (tcx - pcx) / (pw / 10.0)
    g_cy = (tcy - pcy) / (ph / 10.0)
    g_w = jnp.log(tw / pw) * 5.0
    g_h = jnp.log(th / ph) * 5.0
    l1 = (jnp.abs(pbt_ref[0, 0] - g_cx)
          + jnp.abs(pbt_ref[0, 1] - g_cy)
          + jnp.abs(pbt_ref[0, 2] - g_w)
          + jnp.abs(pbt_ref[0, 3] - g_h))
    ls_ref[0, 0, 0] = jnp.sum(l1 * posf)
    np_ref[0, 0, 0] = jnp.sum(posf)

    cls_pos = jnp.zeros((), jnp.float32)
    for r in range(_SUB):
        r0 = r * _LANES
        rows = min(_LANES, _N_PRIORS - r0)
        st = s_ref[0, r0:r0 + rows, :].T
        m = jnp.max(st, axis=0, keepdims=True)
        e = jnp.exp(st - m)
        lse = m + jnp.log(jnp.sum(e, axis=0, keepdims=True))
        sio = lax.broadcasted_iota(jnp.int32, (_N_CLASSES, rows), 0)
        ci = cls[r:r + 1, 0:rows]
        strue = jnp.sum(jnp.where(sio == ci.astype(jnp.int32), st, 0.0),
                        axis=0, keepdims=True)
        ce = lse - strue
        pf = posf[r:r + 1, 0:rows]
        cls_pos = cls_pos + jnp.sum(ce * pf)
        neg_ref[0, 0, r0:r0 + rows] = (ce * (1.0 - pf))[0]
    cp_ref[0, 0, 0] = cls_pos


def _mine_kernel(neg_ref, np_ref, ls_ref, cp_ref, out_ref):
    neg = neg_ref[...]
    npos = jnp.sum(np_ref[...])
    cls_pos = jnp.sum(cp_ref[...])
    k = jnp.minimum(jnp.float32(_NEG_POS_RATIO) * npos, jnp.float32(_N_PRIORS))

    lo = jnp.zeros((_BATCH, 1), jnp.int32)
    hi = jnp.full((_BATCH, 1), jnp.int32(0x7F800000))

    def body(_, carry):
        lo, hi = carry
        mid = lo + ((hi - lo) >> 1)
        midf = lax.bitcast_convert_type(mid, jnp.float32)
        cnt = jnp.sum((neg >= midf).astype(jnp.float32), axis=1, keepdims=True)
        p = cnt >= k
        return jnp.where(p, mid, lo), jnp.where(p, hi, mid)

    lo, hi = lax.fori_loop(0, 31, body, (lo, hi))
    v = lax.bitcast_convert_type(lo, jnp.float32)
    gt = (neg > v).astype(jnp.float32)
    cnt_gt = jnp.sum(gt, axis=1, keepdims=True)
    sum_gt = jnp.sum(neg * gt, axis=1, keepdims=True)
    cls_hard = jnp.sum(sum_gt + (k - cnt_gt) * v)

    loc_loss = jnp.sum(ls_ref[...]) / (npos * 4.0)
    out_ref[0, 0] = loc_loss + (cls_pos + cls_hard) / npos


def kernel(pred_boxes, pred_scores, true_boxes, true_classes, pboxes):
    b, p = _BATCH, _N_PRIORS
    pad = _P_PAD - p
    pad_vals = jnp.tile(jnp.array([[50.0], [50.0], [1.0], [1.0]],
                                  jnp.float32), (1, pad))
    pc_t = jnp.concatenate([jnp.transpose(pboxes, (1, 0)), pad_vals],
                           axis=1).reshape(4, _SUB, _LANES)
    pb_t = jnp.concatenate(
        [jnp.transpose(pred_boxes, (0, 2, 1)),
         jnp.zeros((b, 4, pad), jnp.float32)], axis=2
    ).reshape(b, 4, _SUB, _LANES)

    neg, npos, locsum, clspos = pl.pallas_call(
        _fused_kernel,
        grid=(b,),
        in_specs=[
            pl.BlockSpec((1, _N_OBJS, 4), lambda i: (i, 0, 0),
                         memory_space=pltpu.SMEM),
            pl.BlockSpec((1, 1, _N_OBJS), lambda i: (i, 0, 0),
                         memory_space=pltpu.SMEM),
            pl.BlockSpec((4, _SUB, _LANES), lambda i: (0, 0, 0)),
            pl.BlockSpec((1, 4, _SUB, _LANES), lambda i: (i, 0, 0, 0)),
            pl.BlockSpec((1, p, _N_CLASSES), lambda i: (i, 0, 0)),
        ],
        out_specs=[
            pl.BlockSpec((1, 1, p), lambda i: (i, 0, 0)),
            pl.BlockSpec((1, 1, 1), lambda i: (i, 0, 0),
                         memory_space=pltpu.SMEM),
            pl.BlockSpec((1, 1, 1), lambda i: (i, 0, 0),
                         memory_space=pltpu.SMEM),
            pl.BlockSpec((1, 1, 1), lambda i: (i, 0, 0),
                         memory_space=pltpu.SMEM),
        ],
        compiler_params=pltpu.CompilerParams(
            dimension_semantics=("parallel",)),
        out_shape=[
            jax.ShapeDtypeStruct((b, 1, p), jnp.float32),
            jax.ShapeDtypeStruct((b, 1, 1), jnp.float32),
            jax.ShapeDtypeStruct((b, 1, 1), jnp.float32),
            jax.ShapeDtypeStruct((b, 1, 1), jnp.float32),
        ],
    )(true_boxes, true_classes[:, None], pc_t, pb_t, pred_scores)

    loss = pl.pallas_call(
        _mine_kernel,
        in_specs=[
            pl.BlockSpec((b, p), lambda: (0, 0)),
            pl.BlockSpec((b, 1, 1), lambda: (0, 0, 0)),
            pl.BlockSpec((b, 1, 1), lambda: (0, 0, 0)),
            pl.BlockSpec((b, 1, 1), lambda: (0, 0, 0)),
        ],
        out_specs=pl.BlockSpec((1, 1), lambda: (0, 0),
                               memory_space=pltpu.SMEM),
        out_shape=jax.ShapeDtypeStruct((1, 1), jnp.float32),
    )(neg.reshape(b, p), npos, locsum, clspos)

    return loss[0, 0]
